# Initial kernel scaffold; baseline (speedup 1.0000x reference)
import jax
import jax.numpy as jnp
from jax.experimental import pallas as pl


def kernel(x, a_indices, e, i, idx_a, idx_b, Wk1, bk1, R1, b1, Wk2, bk2, R2, b2, Wk3, bk3, R3, b3, Wk4, bk4, R4, b4, Wk5, bk5, R5, b5, Wd, bd):
    raise NotImplementedError("write your pallas kernel here")


# speedup vs baseline: 5.0817x; 5.0817x over previous
import functools

import jax
import jax.numpy as jnp
from jax import lax
from jax.experimental import pallas as pl
from jax.experimental.pallas import tpu as pltpu
from jax.experimental.pallas import tpu_sc as plsc

N_NODES = 2048
N_EDGES = 4096
N_PREF = 1024
FINS = [32, 256, 128, 64, 32]
FOUTS = [256, 128, 64, 32, 16]

NSUB = 16
ECH = 128
EPT = N_EDGES // NSUB
NCHUNK = EPT // ECH
NPT = N_NODES // NSUB
PPT = N_PREF // NSUB

_MESH = plsc.VectorSubcoreMesh(core_axis_name="c", subcore_axis_name="s",
                               num_cores=1)
_SC_PARAMS = pltpu.CompilerParams(use_tc_tiling_on_sc=False)


def _sc_gather(n_table_rows, d):
    @functools.partial(
        pl.kernel, mesh=_MESH, compiler_params=_SC_PARAMS,
        out_type=jax.ShapeDtypeStruct((N_EDGES, d), jnp.float32),
        scratch_types=[
            pltpu.VMEM((NCHUNK, ECH), jnp.int32),
            pltpu.VMEM((ECH, d), jnp.float32),
            pltpu.SemaphoreType.DMA,
        ],
    )
    def k(table, idx3, out, idx_v, buf_v, sem):
        s = lax.axis_index("s")
        pltpu.sync_copy(idx3.at[s], idx_v)
        for j in range(NCHUNK):
            pltpu.async_copy(table.at[idx_v.at[j]], buf_v, sem).wait()
            pltpu.sync_copy(buf_v, out.at[pl.ds((s * NCHUNK + j) * ECH, ECH)])

    return k


def _sc_layer(fout, last):
    outs = [jax.ShapeDtypeStruct((N_NODES, fout), jnp.float32)]
    if not last:
        outs.append(jax.ShapeDtypeStruct((N_EDGES, fout), jnp.float32))

    @functools.partial(
        pl.kernel, mesh=_MESH, compiler_params=_SC_PARAMS,
        out_type=tuple(outs),
        scratch_types=[
            pltpu.VMEM_SHARED((N_NODES, fout), jnp.float32),
            pltpu.VMEM((NCHUNK, ECH), jnp.int32),
            pltpu.VMEM((ECH, fout), jnp.float32),
            pltpu.VMEM((NPT, fout), jnp.float32),
            pltpu.SemaphoreType.DMA,
        ],
    )
    def k(*refs):
        if last:
            msgs, dst3, root, hpre = refs[:4]
            acc, idx_v, buf_v, node_v, sem = refs[4:]
        else:
            msgs, dst3, root, src3, hpre, xnext = refs[:6]
            acc, idx_v, buf_v, node_v, sem = refs[6:]
        s = lax.axis_index("s")
        nb = s * NPT
        pltpu.sync_copy(root.at[pl.ds(nb, NPT)], node_v)
        pltpu.sync_copy(node_v, acc.at[pl.ds(nb, NPT)])
        pltpu.sync_copy(dst3.at[s], idx_v)
        plsc.subcore_barrier()
        for j in range(NCHUNK):
            pltpu.sync_copy(msgs.at[pl.ds((s * NCHUNK + j) * ECH, ECH)], buf_v)
            pltpu.sync_copy(buf_v, acc.at[idx_v.at[j]], add=True)
        plsc.subcore_barrier()
        pltpu.sync_copy(acc.at[pl.ds(nb, NPT)], node_v)
        pltpu.sync_copy(node_v, hpre.at[pl.ds(nb, NPT)])
        if not last:
            pltpu.sync_copy(src3.at[s], idx_v)
            plsc.subcore_barrier()
            for j in range(NCHUNK):
                pltpu.async_copy(hpre.at[idx_v.at[j]], buf_v, sem).wait()
                pltpu.sync_copy(
                    buf_v, xnext.at[pl.ds((s * NCHUNK + j) * ECH, ECH)])

    return k


@functools.partial(
    pl.kernel, mesh=_MESH, compiler_params=_SC_PARAMS,
    out_type=jax.ShapeDtypeStruct((N_PREF, 16), jnp.float32),
    scratch_types=[
        pltpu.VMEM((2, PPT), jnp.int32),
        pltpu.VMEM((PPT, 16), jnp.float32),
        pltpu.VMEM((PPT, 16), jnp.float32),
        pltpu.SemaphoreType.DMA,
    ],
)
def _sc_pref(z16, iab3, out, idx_v, a_v, b_v, sem):
    s = lax.axis_index("s")
    pltpu.sync_copy(iab3.at[s], idx_v)
    pltpu.async_copy(z16.at[idx_v.at[0]], a_v, sem).wait()
    pltpu.async_copy(z16.at[idx_v.at[1]], b_v, sem).wait()
    for r in range(PPT):
        a_v[r] = b_v[r] - a_v[r]
    pltpu.sync_copy(a_v, out.at[pl.ds(s * PPT, PPT)])


def _tc_layer(fin, fout, first):
    def body(x_ref, h_ref, ep_ref, w_ref, r_ref, b_ref, msgs_ref, root_ref):
        X = x_ref[...]
        H = h_ref[...]
        if not first:
            X = jnp.maximum(X, 0.0)
            H = jnp.maximum(H, 0.0)
        acc = ep_ref[:, 0:1] * jnp.dot(X, w_ref[0],
                                       preferred_element_type=jnp.float32)
        for c in range(1, 5):
            acc += ep_ref[:, c:c + 1] * jnp.dot(
                X, w_ref[c], preferred_element_type=jnp.float32)
        msgs_ref[...] = acc
        root_ref[...] = jnp.dot(H, r_ref[...],
                                preferred_element_type=jnp.float32) + b_ref[...]

    return pl.pallas_call(
        body,
        out_shape=(jax.ShapeDtypeStruct((N_EDGES, fout), jnp.float32),
                   jax.ShapeDtypeStruct((N_NODES, fout), jnp.float32)),
    )


def _tc_final():
    def body(h_ref, wd_ref, bd_ref, z_ref):
        H = jnp.maximum(h_ref[...], 0.0)
        z = jnp.dot(H, wd_ref[...],
                    preferred_element_type=jnp.float32) + bd_ref[...]
        z = jnp.maximum(z, 0.0)
        z_ref[...] = jnp.pad(z, ((0, 0), (0, 8)))

    return pl.pallas_call(
        body, out_shape=jax.ShapeDtypeStruct((N_NODES, 16), jnp.float32))


def kernel(x, a_indices, e, i, idx_a, idx_b,
           Wk1, bk1, R1, b1, Wk2, bk2, R2, b2, Wk3, bk3, R3, b3,
           Wk4, bk4, R4, b4, Wk5, bk5, R5, b5, Wd, bd):
    del i
    x32 = x.astype(jnp.float32)[:, :32]
    e = e.astype(jnp.float32)
    src3 = a_indices[:, 0].reshape(NSUB, NCHUNK, ECH)
    dst3 = a_indices[:, 1].reshape(NSUB, NCHUNK, ECH)
    ep = jnp.concatenate([e, jnp.ones((N_EDGES, 1), jnp.float32)], axis=1)

    Wks = [Wk1, Wk2, Wk3, Wk4, Wk5]
    bks = [bk1, bk2, bk3, bk4, bk5]
    Rs = [R1, R2, R3, R4, R5]
    bs = [b1, b2, b3, b4, b5]
    W5s, b2ds = [], []
    for l in range(5):
        fin, fout = FINS[l], FOUTS[l]
        W5s.append(jnp.concatenate(
            [Wks[l].reshape(4, fin, fout), bks[l].reshape(1, fin, fout)],
            axis=0))
        b2ds.append(bs[l].reshape(1, fout))

    X = _sc_gather(N_NODES, 32)(x32, src3)
    h = x32
    for l in range(5):
        msgs, root = _tc_layer(FINS[l], FOUTS[l], first=(l == 0))(
            X, h, ep, W5s[l], Rs[l], b2ds[l])
        if l < 4:
            h, X = _sc_layer(FOUTS[l], last=False)(msgs, dst3, root, src3)
        else:
            (h,) = _sc_layer(FOUTS[l], last=True)(msgs, dst3, root)

    z16 = _tc_final()(h, Wd, bd.reshape(1, 8))

    iab3 = jnp.stack([idx_a.reshape(NSUB, PPT), idx_b.reshape(NSUB, PPT)],
                     axis=1)
    out16 = _sc_pref(z16, iab3)
    return out16[:, :8]


# speedup vs baseline: 5.6487x; 1.1116x over previous
import functools

import jax
import jax.numpy as jnp
from jax import lax
from jax.experimental import pallas as pl
from jax.experimental.pallas import tpu as pltpu
from jax.experimental.pallas import tpu_sc as plsc

N_NODES = 2048
N_EDGES = 4096
N_PREF = 1024
FINS = [32, 256, 128, 64, 32]
FOUTS = [256, 128, 64, 32, 16]

NSUB = 16
ECH = 128
EPT = N_EDGES // NSUB
NCHUNK = EPT // ECH
NPT = N_NODES // NSUB
PPT = N_PREF // NSUB

_MESH = plsc.VectorSubcoreMesh(core_axis_name="c", subcore_axis_name="s",
                               num_cores=1)
_SC_PARAMS = pltpu.CompilerParams(use_tc_tiling_on_sc=False)


def _sc_gather(n_table_rows, d):
    @functools.partial(
        pl.kernel, mesh=_MESH, compiler_params=_SC_PARAMS,
        out_type=jax.ShapeDtypeStruct((N_EDGES, d), jnp.float32),
        scratch_types=[
            pltpu.VMEM((NCHUNK, ECH), jnp.int32),
            pltpu.VMEM((EPT, d), jnp.float32),
            pltpu.SemaphoreType.DMA,
        ],
    )
    def k(table, idx3, out, idx_v, buf_v, sem):
        s = lax.axis_index("s")
        pltpu.sync_copy(idx3.at[s], idx_v)
        cs = [pltpu.async_copy(table.at[idx_v.at[j]],
                               buf_v.at[pl.ds(j * ECH, ECH)], sem)
              for j in range(NCHUNK)]
        for c in cs:
            c.wait()
        pltpu.sync_copy(buf_v, out.at[pl.ds(s * EPT, EPT)])

    return k


def _sc_layer(fout, last):
    outs = [jax.ShapeDtypeStruct((N_NODES, fout), jnp.float32)]
    if not last:
        outs.append(jax.ShapeDtypeStruct((N_EDGES, fout), jnp.float32))

    @functools.partial(
        pl.kernel, mesh=_MESH, compiler_params=_SC_PARAMS,
        out_type=tuple(outs),
        scratch_types=[
            pltpu.VMEM_SHARED((N_NODES, fout), jnp.float32),
            pltpu.VMEM((NCHUNK, ECH), jnp.int32),
            pltpu.VMEM((EPT, fout), jnp.float32),
            pltpu.SemaphoreType.DMA,
        ],
    )
    def k(*refs):
        if last:
            msgs, dst3, root, hpre = refs[:4]
            acc, idx_v, buf_v, sem = refs[4:]
        else:
            msgs, dst3, root, src3, hpre, xnext = refs[:6]
            acc, idx_v, buf_v, sem = refs[6:]
        s = lax.axis_index("s")
        nb = s * NPT
        eb = s * EPT
        ca = pltpu.async_copy(root.at[pl.ds(nb, NPT)], acc.at[pl.ds(nb, NPT)],
                              sem)
        cb = pltpu.async_copy(dst3.at[s], idx_v, sem)
        cc = pltpu.async_copy(msgs.at[pl.ds(eb, EPT)], buf_v, sem)
        ca.wait()
        cb.wait()
        cc.wait()
        plsc.subcore_barrier()
        for j in range(NCHUNK):
            pltpu.sync_copy(buf_v.at[pl.ds(j * ECH, ECH)], acc.at[idx_v.at[j]],
                            add=True)
        plsc.subcore_barrier()
        cd = pltpu.async_copy(acc.at[pl.ds(nb, NPT)], hpre.at[pl.ds(nb, NPT)],
                              sem)
        if not last:
            pltpu.sync_copy(src3.at[s], idx_v)
            cd.wait()
            plsc.subcore_barrier()
            cs = [pltpu.async_copy(hpre.at[idx_v.at[j]],
                                   buf_v.at[pl.ds(j * ECH, ECH)], sem)
                  for j in range(NCHUNK)]
            for c in cs:
                c.wait()
            pltpu.sync_copy(buf_v, xnext.at[pl.ds(eb, EPT)])
        else:
            cd.wait()

    return k


@functools.partial(
    pl.kernel, mesh=_MESH, compiler_params=_SC_PARAMS,
    out_type=jax.ShapeDtypeStruct((N_PREF, 16), jnp.float32),
    scratch_types=[
        pltpu.VMEM((2, PPT), jnp.int32),
        pltpu.VMEM((PPT, 16), jnp.float32),
        pltpu.VMEM((PPT, 16), jnp.float32),
        pltpu.SemaphoreType.DMA,
    ],
)
def _sc_pref(z16, iab3, out, idx_v, a_v, b_v, sem):
    s = lax.axis_index("s")
    pltpu.sync_copy(iab3.at[s], idx_v)
    c0 = pltpu.async_copy(z16.at[idx_v.at[0]], a_v, sem)
    c1 = pltpu.async_copy(z16.at[idx_v.at[1]], b_v, sem)
    c0.wait()
    c1.wait()
    for r in range(PPT):
        a_v[r] = b_v[r] - a_v[r]
    pltpu.sync_copy(a_v, out.at[pl.ds(s * PPT, PPT)])


def _tc_layer(fin, fout, first):
    def body(x_ref, h_ref, ep_ref, w_ref, r_ref, b_ref, msgs_ref, root_ref):
        X = x_ref[...]
        H = h_ref[...]
        if not first:
            X = jnp.maximum(X, 0.0)
            H = jnp.maximum(H, 0.0)
        acc = ep_ref[:, 0:1] * jnp.dot(X, w_ref[0],
                                       preferred_element_type=jnp.float32)
        for c in range(1, 5):
            acc += ep_ref[:, c:c + 1] * jnp.dot(
                X, w_ref[c], preferred_element_type=jnp.float32)
        msgs_ref[...] = acc
        root_ref[...] = jnp.dot(H, r_ref[...],
                                preferred_element_type=jnp.float32) + b_ref[...]

    return pl.pallas_call(
        body,
        out_shape=(jax.ShapeDtypeStruct((N_EDGES, fout), jnp.float32),
                   jax.ShapeDtypeStruct((N_NODES, fout), jnp.float32)),
    )


def _tc_final():
    def body(h_ref, wd_ref, bd_ref, z_ref):
        H = jnp.maximum(h_ref[...], 0.0)
        z = jnp.dot(H, wd_ref[...],
                    preferred_element_type=jnp.float32) + bd_ref[...]
        z = jnp.maximum(z, 0.0)
        z_ref[...] = jnp.pad(z, ((0, 0), (0, 8)))

    return pl.pallas_call(
        body, out_shape=jax.ShapeDtypeStruct((N_NODES, 16), jnp.float32))


def kernel(x, a_indices, e, i, idx_a, idx_b,
           Wk1, bk1, R1, b1, Wk2, bk2, R2, b2, Wk3, bk3, R3, b3,
           Wk4, bk4, R4, b4, Wk5, bk5, R5, b5, Wd, bd):
    del i
    x32 = x.astype(jnp.float32)[:, :32]
    e = e.astype(jnp.float32)
    src3 = a_indices[:, 0].reshape(NSUB, NCHUNK, ECH)
    dst3 = a_indices[:, 1].reshape(NSUB, NCHUNK, ECH)
    ep = jnp.concatenate([e, jnp.ones((N_EDGES, 1), jnp.float32)], axis=1)

    Wks = [Wk1, Wk2, Wk3, Wk4, Wk5]
    bks = [bk1, bk2, bk3, bk4, bk5]
    Rs = [R1, R2, R3, R4, R5]
    bs = [b1, b2, b3, b4, b5]
    W5s, b2ds = [], []
    for l in range(5):
        fin, fout = FINS[l], FOUTS[l]
        W5s.append(jnp.concatenate(
            [Wks[l].reshape(4, fin, fout), bks[l].reshape(1, fin, fout)],
            axis=0))
        b2ds.append(bs[l].reshape(1, fout))

    X = _sc_gather(N_NODES, 32)(x32, src3)
    h = x32
    for l in range(5):
        msgs, root = _tc_layer(FINS[l], FOUTS[l], first=(l == 0))(
            X, h, ep, W5s[l], Rs[l], b2ds[l])
        if l < 4:
            h, X = _sc_layer(FOUTS[l], last=False)(msgs, dst3, root, src3)
        else:
            (h,) = _sc_layer(FOUTS[l], last=True)(msgs, dst3, root)

    z16 = _tc_final()(h, Wd, bd.reshape(1, 8))

    iab3 = jnp.stack([idx_a.reshape(NSUB, PPT), idx_b.reshape(NSUB, PPT)],
                     axis=1)
    out16 = _sc_pref(z16, iab3)
    return out16[:, :8]


# speedup vs baseline: 5.8135x; 1.0292x over previous
import functools

import jax
import jax.numpy as jnp
from jax import lax
from jax.experimental import pallas as pl
from jax.experimental.pallas import tpu as pltpu
from jax.experimental.pallas import tpu_sc as plsc

N_NODES = 2048
N_EDGES = 4096
N_PREF = 1024
FINS = [32, 256, 128, 64, 32]
FOUTS = [256, 128, 64, 32, 16]

NSUB = 16
ECH = 128
EPT = N_EDGES // NSUB
NCHUNK = EPT // ECH
NPT = N_NODES // NSUB
PPT = N_PREF // NSUB

_MESH = plsc.VectorSubcoreMesh(core_axis_name="c", subcore_axis_name="s",
                               num_cores=1)
_SC_PARAMS = pltpu.CompilerParams(use_tc_tiling_on_sc=False)


def _sc_gather(n_table_rows, d):
    @functools.partial(
        pl.kernel, mesh=_MESH, compiler_params=_SC_PARAMS,
        out_type=jax.ShapeDtypeStruct((N_EDGES, d), jnp.float32),
        scratch_types=[
            pltpu.VMEM((NCHUNK, ECH), jnp.int32),
            pltpu.VMEM((EPT, d), jnp.float32),
            pltpu.SemaphoreType.DMA,
        ],
    )
    def k(table, idx3, out, idx_v, buf_v, sem):
        s = lax.axis_index("s")
        pltpu.sync_copy(idx3.at[s], idx_v)
        cs = [pltpu.async_copy(table.at[idx_v.at[j]],
                               buf_v.at[pl.ds(j * ECH, ECH)], sem)
              for j in range(NCHUNK)]
        for c in cs:
            c.wait()
        pltpu.sync_copy(buf_v, out.at[pl.ds(s * EPT, EPT)])

    return k


def _sc_layer(fout, gchunks):
    outs = (jax.ShapeDtypeStruct((N_NODES, fout), jnp.float32),
            jax.ShapeDtypeStruct((NSUB * gchunks * ECH, fout), jnp.float32))

    @functools.partial(
        pl.kernel, mesh=_MESH, compiler_params=_SC_PARAMS,
        out_type=outs,
        scratch_types=[
            pltpu.VMEM_SHARED((N_NODES, fout), jnp.float32),
            pltpu.VMEM((max(NCHUNK, gchunks), ECH), jnp.int32),
            pltpu.VMEM((EPT, fout), jnp.float32),
            pltpu.SemaphoreType.DMA,
        ],
    )
    def k(*refs):
        msgs, dst3, root, src3, hpre, xnext = refs[:6]
        acc, idx_v, buf_v, sem = refs[6:]
        s = lax.axis_index("s")
        nb = s * NPT
        eb = s * EPT
        gpt = gchunks * ECH
        ca = pltpu.async_copy(root.at[pl.ds(nb, NPT)], acc.at[pl.ds(nb, NPT)],
                              sem)
        cb = pltpu.async_copy(dst3.at[s], idx_v, sem)
        cc = pltpu.async_copy(msgs.at[pl.ds(eb, EPT)], buf_v, sem)
        ca.wait()
        cb.wait()
        cc.wait()
        plsc.subcore_barrier()
        for j in range(NCHUNK):
            pltpu.sync_copy(buf_v.at[pl.ds(j * ECH, ECH)], acc.at[idx_v.at[j]],
                            add=True)
        plsc.subcore_barrier()
        cd = pltpu.async_copy(acc.at[pl.ds(nb, NPT)], hpre.at[pl.ds(nb, NPT)],
                              sem)
        pltpu.sync_copy(src3.at[s], idx_v.at[pl.ds(0, gchunks)])
        cd.wait()
        plsc.subcore_barrier()
        cs = [pltpu.async_copy(hpre.at[idx_v.at[j]],
                               buf_v.at[pl.ds(j * ECH, ECH)], sem)
              for j in range(gchunks)]
        for c in cs:
            c.wait()
        pltpu.sync_copy(buf_v.at[pl.ds(0, gpt)], xnext.at[pl.ds(s * gpt, gpt)])

    return k


@functools.partial(
    pl.kernel, mesh=_MESH, compiler_params=_SC_PARAMS,
    out_type=jax.ShapeDtypeStruct((N_PREF, 16), jnp.float32),
    scratch_types=[
        pltpu.VMEM((2, PPT), jnp.int32),
        pltpu.VMEM((PPT, 16), jnp.float32),
        pltpu.VMEM((PPT, 16), jnp.float32),
        pltpu.SemaphoreType.DMA,
    ],
)
def _sc_pref(z16, iab3, out, idx_v, a_v, b_v, sem):
    s = lax.axis_index("s")
    pltpu.sync_copy(iab3.at[s], idx_v)
    c0 = pltpu.async_copy(z16.at[idx_v.at[0]], a_v, sem)
    c1 = pltpu.async_copy(z16.at[idx_v.at[1]], b_v, sem)
    c0.wait()
    c1.wait()
    for r in range(PPT):
        a_v[r] = b_v[r] - a_v[r]
    pltpu.sync_copy(a_v, out.at[pl.ds(s * PPT, PPT)])


def _tc_layer(fin, fout, first):
    def body(x_ref, h_ref, ep_ref, w_ref, r_ref, b_ref, msgs_ref, root_ref):
        X = x_ref[...]
        H = h_ref[...]
        if not first:
            X = jnp.maximum(X, 0.0)
            H = jnp.maximum(H, 0.0)
        acc = ep_ref[:, 0:1] * jnp.dot(X, w_ref[0],
                                       preferred_element_type=jnp.float32)
        for c in range(1, 5):
            acc += ep_ref[:, c:c + 1] * jnp.dot(
                X, w_ref[c], preferred_element_type=jnp.float32)
        msgs_ref[...] = acc
        root_ref[...] = jnp.dot(H, r_ref[...],
                                preferred_element_type=jnp.float32) + b_ref[...]

    return pl.pallas_call(
        body,
        out_shape=(jax.ShapeDtypeStruct((N_EDGES, fout), jnp.float32),
                   jax.ShapeDtypeStruct((N_NODES, fout), jnp.float32)),
    )


def _tc_final():
    def body(g_ref, wd_ref, bd_ref, o_ref):
        G = jnp.maximum(g_ref[...], 0.0)
        z = jnp.dot(G, wd_ref[...],
                    preferred_element_type=jnp.float32) + bd_ref[...]
        z = jnp.maximum(z, 0.0)
        o_ref[...] = z[N_PREF:, :] - z[:N_PREF, :]

    return pl.pallas_call(
        body, out_shape=jax.ShapeDtypeStruct((N_PREF, 8), jnp.float32))


def kernel(x, a_indices, e, i, idx_a, idx_b,
           Wk1, bk1, R1, b1, Wk2, bk2, R2, b2, Wk3, bk3, R3, b3,
           Wk4, bk4, R4, b4, Wk5, bk5, R5, b5, Wd, bd):
    del i
    x32 = x.astype(jnp.float32)[:, :32]
    e = e.astype(jnp.float32)
    src3 = a_indices[:, 0].reshape(NSUB, NCHUNK, ECH)
    dst3 = a_indices[:, 1].reshape(NSUB, NCHUNK, ECH)
    ep = jnp.concatenate([e, jnp.ones((N_EDGES, 1), jnp.float32)], axis=1)

    Wks = [Wk1, Wk2, Wk3, Wk4, Wk5]
    bks = [bk1, bk2, bk3, bk4, bk5]
    Rs = [R1, R2, R3, R4, R5]
    bs = [b1, b2, b3, b4, b5]
    W5s, b2ds = [], []
    for l in range(5):
        fin, fout = FINS[l], FOUTS[l]
        W5s.append(jnp.concatenate(
            [Wks[l].reshape(4, fin, fout), bks[l].reshape(1, fin, fout)],
            axis=0))
        b2ds.append(bs[l].reshape(1, fout))

    pref3 = jnp.concatenate([idx_a, idx_b]).reshape(NSUB, 1, ECH)

    X = _sc_gather(N_NODES, 32)(x32, src3)
    h = x32
    for l in range(5):
        msgs, root = _tc_layer(FINS[l], FOUTS[l], first=(l == 0))(
            X, h, ep, W5s[l], Rs[l], b2ds[l])
        gidx = src3 if l < 4 else pref3
        h, X = _sc_layer(FOUTS[l], gchunks=gidx.shape[1])(
            msgs, dst3, root, gidx)

    return _tc_final()(X, Wd, bd.reshape(1, 8))


# speedup vs baseline: 6.0310x; 1.0374x over previous
import functools

import jax
import jax.numpy as jnp
from jax import lax
from jax.experimental import pallas as pl
from jax.experimental.pallas import tpu as pltpu
from jax.experimental.pallas import tpu_sc as plsc

N_NODES = 2048
N_EDGES = 4096
N_PREF = 1024
FINS = [32, 256, 128, 64, 32]
FOUTS = [256, 128, 64, 32, 16]

NSUB = 16
ECH = 128
EPT = N_EDGES // NSUB
NCHUNK = EPT // ECH
NPT = N_NODES // NSUB
PPT = N_PREF // NSUB

_MESH = plsc.VectorSubcoreMesh(core_axis_name="c", subcore_axis_name="s",
                               num_cores=1)
_SC_PARAMS = pltpu.CompilerParams(use_tc_tiling_on_sc=False)


def _sc_gather(n_table_rows, d):
    @functools.partial(
        pl.kernel, mesh=_MESH, compiler_params=_SC_PARAMS,
        out_type=jax.ShapeDtypeStruct((N_EDGES, d), jnp.float32),
        scratch_types=[
            pltpu.VMEM((NCHUNK, ECH), jnp.int32),
            pltpu.VMEM((EPT, d), jnp.float32),
            pltpu.SemaphoreType.DMA,
        ],
    )
    def k(table, idx3, out, idx_v, buf_v, sem):
        s = lax.axis_index("s")
        pltpu.sync_copy(idx3.at[s], idx_v)
        cs = [pltpu.async_copy(table.at[idx_v.at[j]],
                               buf_v.at[pl.ds(j * ECH, ECH)], sem)
              for j in range(NCHUNK)]
        for c in cs:
            c.wait()
        pltpu.sync_copy(buf_v, out.at[pl.ds(s * EPT, EPT)])

    return k


def _sc_layer(fout, gchunks):
    outs = (jax.ShapeDtypeStruct((N_NODES, fout), jnp.float32),
            jax.ShapeDtypeStruct((NSUB * gchunks * ECH, fout), jnp.float32))

    @functools.partial(
        pl.kernel, mesh=_MESH, compiler_params=_SC_PARAMS,
        out_type=outs,
        scratch_types=[
            pltpu.VMEM_SHARED((N_NODES, fout), jnp.float32),
            pltpu.VMEM((max(NCHUNK, gchunks), ECH), jnp.int32),
            pltpu.VMEM((EPT, fout), jnp.float32),
            pltpu.SemaphoreType.DMA,
        ],
    )
    def k(*refs):
        msgs, dst3, root, src3, hpre, xnext = refs[:6]
        acc, idx_v, buf_v, sem = refs[6:]
        s = lax.axis_index("s")
        nb = s * NPT
        eb = s * EPT
        gpt = gchunks * ECH
        ca = pltpu.async_copy(root.at[pl.ds(nb, NPT)], acc.at[pl.ds(nb, NPT)],
                              sem)
        cb = pltpu.async_copy(dst3.at[s], idx_v, sem)
        cc = pltpu.async_copy(msgs.at[pl.ds(eb, EPT)], buf_v, sem)
        ca.wait()
        cb.wait()
        cc.wait()
        plsc.subcore_barrier()
        for j in range(NCHUNK):
            pltpu.sync_copy(buf_v.at[pl.ds(j * ECH, ECH)], acc.at[idx_v.at[j]],
                            add=True)
        plsc.subcore_barrier()
        cd = pltpu.async_copy(acc.at[pl.ds(nb, NPT)], hpre.at[pl.ds(nb, NPT)],
                              sem)
        pltpu.sync_copy(src3.at[s], idx_v.at[pl.ds(0, gchunks)])
        cd.wait()
        plsc.subcore_barrier()
        cs = [pltpu.async_copy(hpre.at[idx_v.at[j]],
                               buf_v.at[pl.ds(j * ECH, ECH)], sem)
              for j in range(gchunks)]
        for c in cs:
            c.wait()
        pltpu.sync_copy(buf_v.at[pl.ds(0, gpt)], xnext.at[pl.ds(s * gpt, gpt)])

    return k


@functools.partial(
    pl.kernel, mesh=_MESH, compiler_params=_SC_PARAMS,
    out_type=jax.ShapeDtypeStruct((N_PREF, 16), jnp.float32),
    scratch_types=[
        pltpu.VMEM((2, PPT), jnp.int32),
        pltpu.VMEM((PPT, 16), jnp.float32),
        pltpu.VMEM((PPT, 16), jnp.float32),
        pltpu.SemaphoreType.DMA,
    ],
)
def _sc_pref(z16, iab3, out, idx_v, a_v, b_v, sem):
    s = lax.axis_index("s")
    pltpu.sync_copy(iab3.at[s], idx_v)
    c0 = pltpu.async_copy(z16.at[idx_v.at[0]], a_v, sem)
    c1 = pltpu.async_copy(z16.at[idx_v.at[1]], b_v, sem)
    c0.wait()
    c1.wait()
    for r in range(PPT):
        a_v[r] = b_v[r] - a_v[r]
    pltpu.sync_copy(a_v, out.at[pl.ds(s * PPT, PPT)])


def _tc_layer(fin, fout, first):
    def body(x_ref, h_ref, ep_ref, w_ref, r_ref, b_ref, msgs_ref, root_ref):
        X = x_ref[...]
        H = h_ref[...]
        if not first:
            X = jnp.maximum(X, 0.0)
            H = jnp.maximum(H, 0.0)
        acc = ep_ref[:, 0:1] * jnp.dot(X, w_ref[0],
                                       preferred_element_type=jnp.float32)
        for c in range(1, 5):
            acc += ep_ref[:, c:c + 1] * jnp.dot(
                X, w_ref[c], preferred_element_type=jnp.float32)
        msgs_ref[...] = acc
        root_ref[...] = jnp.dot(H, r_ref[...],
                                preferred_element_type=jnp.float32) + b_ref[...]

    return pl.pallas_call(
        body,
        out_shape=(jax.ShapeDtypeStruct((N_EDGES, fout), jnp.float32),
                   jax.ShapeDtypeStruct((N_NODES, fout), jnp.float32)),
    )


def _tc_last():
    def body(x_ref, h_ref, ep_ref, w_ref, r_ref, b_ref, dst_ref,
             ia_ref, ib_ref, wd_ref, bd_ref, o_ref):
        X = jnp.maximum(x_ref[...], 0.0)
        H = jnp.maximum(h_ref[...], 0.0)
        acc = ep_ref[:, 0:1] * jnp.dot(X, w_ref[0],
                                       preferred_element_type=jnp.float32)
        for c in range(1, 5):
            acc += ep_ref[:, c:c + 1] * jnp.dot(
                X, w_ref[c], preferred_element_type=jnp.float32)
        root = jnp.dot(H, r_ref[...],
                       preferred_element_type=jnp.float32) + b_ref[...]
        ids = lax.broadcasted_iota(jnp.int32, (N_NODES, N_EDGES), 0)
        oh = jnp.where(dst_ref[...] == ids, 1.0, 0.0)
        hpre = jnp.dot(oh, acc, preferred_element_type=jnp.float32) + root
        z = jnp.dot(jnp.maximum(hpre, 0.0), wd_ref[...],
                    preferred_element_type=jnp.float32) + bd_ref[...]
        z = jnp.maximum(z, 0.0)
        idn = lax.broadcasted_iota(jnp.int32, (N_PREF, N_NODES), 1)
        pd = (jnp.where(ib_ref[...] == idn, 1.0, 0.0)
              - jnp.where(ia_ref[...] == idn, 1.0, 0.0))
        o_ref[...] = jnp.dot(pd, z, preferred_element_type=jnp.float32)

    return pl.pallas_call(
        body, out_shape=jax.ShapeDtypeStruct((N_PREF, 8), jnp.float32))


def kernel(x, a_indices, e, i, idx_a, idx_b,
           Wk1, bk1, R1, b1, Wk2, bk2, R2, b2, Wk3, bk3, R3, b3,
           Wk4, bk4, R4, b4, Wk5, bk5, R5, b5, Wd, bd):
    del i
    x32 = x.astype(jnp.float32)[:, :32]
    e = e.astype(jnp.float32)
    src3 = a_indices[:, 0].reshape(NSUB, NCHUNK, ECH)
    dst3 = a_indices[:, 1].reshape(NSUB, NCHUNK, ECH)
    ep = jnp.concatenate([e, jnp.ones((N_EDGES, 1), jnp.float32)], axis=1)

    Wks = [Wk1, Wk2, Wk3, Wk4, Wk5]
    bks = [bk1, bk2, bk3, bk4, bk5]
    Rs = [R1, R2, R3, R4, R5]
    bs = [b1, b2, b3, b4, b5]
    W5s, b2ds = [], []
    for l in range(5):
        fin, fout = FINS[l], FOUTS[l]
        W5s.append(jnp.concatenate(
            [Wks[l].reshape(4, fin, fout), bks[l].reshape(1, fin, fout)],
            axis=0))
        b2ds.append(bs[l].reshape(1, fout))

    X = _sc_gather(N_NODES, 32)(x32, src3)
    h = x32
    for l in range(4):
        msgs, root = _tc_layer(FINS[l], FOUTS[l], first=(l == 0))(
            X, h, ep, W5s[l], Rs[l], b2ds[l])
        h, X = _sc_layer(FOUTS[l], gchunks=NCHUNK)(msgs, dst3, root, src3)

    return _tc_last()(
        X, h, ep, W5s[4], Rs[4], b2ds[4],
        a_indices[:, 1].reshape(1, N_EDGES),
        idx_a.reshape(N_PREF, 1), idx_b.reshape(N_PREF, 1),
        Wd, bd.reshape(1, 8))


# speedup vs baseline: 6.1246x; 1.0155x over previous
import functools

import jax
import jax.numpy as jnp
from jax import lax
from jax.experimental import pallas as pl
from jax.experimental.pallas import tpu as pltpu
from jax.experimental.pallas import tpu_sc as plsc

N_NODES = 2048
N_EDGES = 4096
N_PREF = 1024
FINS = [32, 256, 128, 64, 32]
FOUTS = [256, 128, 64, 32, 16]

NSUB = 16
ECH = 128
EPT = N_EDGES // NSUB
NCHUNK = EPT // ECH
NPT = N_NODES // NSUB
PPT = N_PREF // NSUB

_MESH = plsc.VectorSubcoreMesh(core_axis_name="c", subcore_axis_name="s",
                               num_cores=1)
_SC_PARAMS = pltpu.CompilerParams(use_tc_tiling_on_sc=False)


def _sc_gather(n_table_rows, d):
    @functools.partial(
        pl.kernel, mesh=_MESH, compiler_params=_SC_PARAMS,
        out_type=jax.ShapeDtypeStruct((N_EDGES, d), jnp.float32),
        scratch_types=[
            pltpu.VMEM((NCHUNK, ECH), jnp.int32),
            pltpu.VMEM((EPT, d), jnp.float32),
            pltpu.SemaphoreType.DMA,
        ],
    )
    def k(table, idx3, out, idx_v, buf_v, sem):
        s = lax.axis_index("s")
        pltpu.sync_copy(idx3.at[s], idx_v)
        cs = [pltpu.async_copy(table.at[idx_v.at[j]],
                               buf_v.at[pl.ds(j * ECH, ECH)], sem)
              for j in range(NCHUNK)]
        for c in cs:
            c.wait()
        pltpu.sync_copy(buf_v, out.at[pl.ds(s * EPT, EPT)])

    return k


def _sc_layer(fout):
    outs = (jax.ShapeDtypeStruct((N_NODES, fout), jnp.float32),
            jax.ShapeDtypeStruct((N_EDGES, fout), jnp.float32))

    @functools.partial(
        pl.kernel, mesh=_MESH, compiler_params=_SC_PARAMS,
        out_type=outs,
        scratch_types=[
            pltpu.VMEM_SHARED((N_NODES, fout), jnp.float32),
            pltpu.VMEM((2 * NCHUNK, ECH), jnp.int32),
            pltpu.VMEM((EPT, fout), jnp.float32),
            pltpu.SemaphoreType.DMA,
        ],
    )
    def k(msgs, idx3, root, hpre, xnext, acc, idx_v, buf_v, sem):
        s = lax.axis_index("s")
        nb = s * NPT
        eb = s * EPT
        ca = pltpu.async_copy(root.at[pl.ds(nb, NPT)], acc.at[pl.ds(nb, NPT)],
                              sem)
        cb = pltpu.async_copy(idx3.at[s], idx_v, sem)
        cc = pltpu.async_copy(msgs.at[pl.ds(eb, EPT)], buf_v, sem)
        ca.wait()
        cb.wait()
        cc.wait()
        plsc.subcore_barrier()
        cs = [pltpu.async_copy(buf_v.at[pl.ds(j * ECH, ECH)],
                               acc.at[idx_v.at[j]], sem, add=True)
              for j in range(NCHUNK)]
        for c in cs:
            c.wait()
        plsc.subcore_barrier()
        cd = pltpu.async_copy(acc.at[pl.ds(nb, NPT)], hpre.at[pl.ds(nb, NPT)],
                              sem)
        cd.wait()
        plsc.subcore_barrier()
        cg = [pltpu.async_copy(hpre.at[idx_v.at[NCHUNK + j]],
                               buf_v.at[pl.ds(j * ECH, ECH)], sem)
              for j in range(NCHUNK)]
        cw = []
        for j in range(NCHUNK):
            cg[j].wait()
            cw.append(pltpu.async_copy(
                buf_v.at[pl.ds(j * ECH, ECH)],
                xnext.at[pl.ds(eb + j * ECH, ECH)], sem))
        for c in cw:
            c.wait()

    return k


def _tc_layer(fin, fout, first):
    def body(x_ref, h_ref, ep_ref, w_ref, r_ref, b_ref, msgs_ref, root_ref):
        X = x_ref[...]
        H = h_ref[...]
        if not first:
            X = jnp.maximum(X, 0.0)
            H = jnp.maximum(H, 0.0)
        acc = ep_ref[:, 0:1] * jnp.dot(X, w_ref[0],
                                       preferred_element_type=jnp.float32)
        for c in range(1, 5):
            acc += ep_ref[:, c:c + 1] * jnp.dot(
                X, w_ref[c], preferred_element_type=jnp.float32)
        msgs_ref[...] = acc
        root_ref[...] = jnp.dot(H, r_ref[...],
                                preferred_element_type=jnp.float32) + b_ref[...]

    return pl.pallas_call(
        body,
        out_shape=(jax.ShapeDtypeStruct((N_EDGES, fout), jnp.float32),
                   jax.ShapeDtypeStruct((N_NODES, fout), jnp.float32)),
    )


def _tc_last():
    def body(x_ref, h_ref, ep_ref, w_ref, r_ref, b_ref, dst_ref,
             ia_ref, ib_ref, wd_ref, bd_ref, o_ref):
        X = jnp.maximum(x_ref[...], 0.0)
        H = jnp.maximum(h_ref[...], 0.0)
        acc = ep_ref[:, 0:1] * jnp.dot(X, w_ref[0],
                                       preferred_element_type=jnp.float32)
        for c in range(1, 5):
            acc += ep_ref[:, c:c + 1] * jnp.dot(
                X, w_ref[c], preferred_element_type=jnp.float32)
        root = jnp.dot(H, r_ref[...],
                       preferred_element_type=jnp.float32) + b_ref[...]
        ids = lax.broadcasted_iota(jnp.int32, (N_NODES, N_EDGES), 0)
        oh = jnp.where(dst_ref[...] == ids, 1.0, 0.0)
        hpre = jnp.dot(oh, acc, preferred_element_type=jnp.float32) + root
        z = jnp.dot(jnp.maximum(hpre, 0.0), wd_ref[...],
                    preferred_element_type=jnp.float32) + bd_ref[...]
        z = jnp.maximum(z, 0.0)
        idn = lax.broadcasted_iota(jnp.int32, (N_PREF, N_NODES), 1)
        pd = (jnp.where(ib_ref[...] == idn, 1.0, 0.0)
              - jnp.where(ia_ref[...] == idn, 1.0, 0.0))
        o_ref[...] = jnp.dot(pd, z, preferred_element_type=jnp.float32)

    return pl.pallas_call(
        body, out_shape=jax.ShapeDtypeStruct((N_PREF, 8), jnp.float32))


def kernel(x, a_indices, e, i, idx_a, idx_b,
           Wk1, bk1, R1, b1, Wk2, bk2, R2, b2, Wk3, bk3, R3, b3,
           Wk4, bk4, R4, b4, Wk5, bk5, R5, b5, Wd, bd):
    del i
    x32 = x.astype(jnp.float32)[:, :32]
    e = e.astype(jnp.float32)
    src3 = a_indices[:, 0].reshape(NSUB, NCHUNK, ECH)
    dst3 = a_indices[:, 1].reshape(NSUB, NCHUNK, ECH)
    ep = jnp.concatenate([e, jnp.ones((N_EDGES, 1), jnp.float32)], axis=1)

    Wks = [Wk1, Wk2, Wk3, Wk4, Wk5]
    bks = [bk1, bk2, bk3, bk4, bk5]
    Rs = [R1, R2, R3, R4, R5]
    bs = [b1, b2, b3, b4, b5]
    W5s, b2ds = [], []
    for l in range(5):
        fin, fout = FINS[l], FOUTS[l]
        W5s.append(jnp.concatenate(
            [Wks[l].reshape(4, fin, fout), bks[l].reshape(1, fin, fout)],
            axis=0))
        b2ds.append(bs[l].reshape(1, fout))

    idx3 = jnp.concatenate([dst3, src3], axis=1)

    X = _sc_gather(N_NODES, 32)(x32, src3)
    h = x32
    for l in range(4):
        msgs, root = _tc_layer(FINS[l], FOUTS[l], first=(l == 0))(
            X, h, ep, W5s[l], Rs[l], b2ds[l])
        h, X = _sc_layer(FOUTS[l])(msgs, idx3, root)

    return _tc_last()(
        X, h, ep, W5s[4], Rs[4], b2ds[4],
        a_indices[:, 1].reshape(1, N_EDGES),
        idx_a.reshape(N_PREF, 1), idx_b.reshape(N_PREF, 1),
        Wd, bd.reshape(1, 8))


# speedup vs baseline: 6.1546x; 1.0049x over previous
import functools

import jax
import jax.numpy as jnp
from jax import lax
from jax.experimental import pallas as pl
from jax.experimental.pallas import tpu as pltpu
from jax.experimental.pallas import tpu_sc as plsc

N_NODES = 2048
N_EDGES = 4096
N_PREF = 1024
FINS = [32, 256, 128, 64, 32]
FOUTS = [256, 128, 64, 32, 16]

NSUB = 16
ECH = 128
EPT = N_EDGES // NSUB
NCHUNK = EPT // ECH
NPT = N_NODES // NSUB

_MESH = plsc.VectorSubcoreMesh(core_axis_name="c", subcore_axis_name="s",
                               num_cores=1)
_SC_PARAMS = pltpu.CompilerParams(use_tc_tiling_on_sc=False)


def _sc_gather(n_table_rows, d):
    @functools.partial(
        pl.kernel, mesh=_MESH, compiler_params=_SC_PARAMS,
        out_type=jax.ShapeDtypeStruct((N_EDGES, d), jnp.float32),
        scratch_types=[
            pltpu.VMEM((NCHUNK, ECH), jnp.int32),
            pltpu.VMEM((EPT, d), jnp.float32),
            pltpu.SemaphoreType.DMA,
        ],
    )
    def k(table, idx3, out, idx_v, buf_v, sem):
        s = lax.axis_index("s")
        pltpu.sync_copy(idx3.at[s], idx_v)
        cs = [pltpu.async_copy(table.at[idx_v.at[j]],
                               buf_v.at[pl.ds(j * ECH, ECH)], sem)
              for j in range(NCHUNK)]
        for c in cs:
            c.wait()
        pltpu.sync_copy(buf_v, out.at[pl.ds(s * EPT, EPT)])

    return k


def _sc_layer(fout):
    outs = (jax.ShapeDtypeStruct((N_NODES, fout), jnp.float32),
            jax.ShapeDtypeStruct((N_EDGES, fout), jnp.float32))

    @functools.partial(
        pl.kernel, mesh=_MESH, compiler_params=_SC_PARAMS,
        out_type=outs,
        scratch_types=[
            pltpu.VMEM_SHARED((N_NODES, fout), jnp.float32),
            pltpu.VMEM((2 * NCHUNK, ECH), jnp.int32),
            pltpu.VMEM((EPT, fout), jnp.float32),
            pltpu.SemaphoreType.DMA,
        ],
    )
    def k(msgs, idx3, root, hpre, xnext, acc, idx_v, buf_v, sem):
        s = lax.axis_index("s")
        nb = s * NPT
        eb = s * EPT
        ca = pltpu.async_copy(root.at[pl.ds(nb, NPT)], acc.at[pl.ds(nb, NPT)],
                              sem)
        cb = pltpu.async_copy(idx3.at[s], idx_v, sem)
        cc = pltpu.async_copy(msgs.at[pl.ds(eb, EPT)], buf_v, sem)
        ca.wait()
        cb.wait()
        cc.wait()
        plsc.subcore_barrier()
        cs = [pltpu.async_copy(buf_v.at[pl.ds(j * ECH, ECH)],
                               acc.at[idx_v.at[j]], sem, add=True)
              for j in range(NCHUNK)]
        for c in cs:
            c.wait()
        plsc.subcore_barrier()
        cd = pltpu.async_copy(acc.at[pl.ds(nb, NPT)], hpre.at[pl.ds(nb, NPT)],
                              sem)
        cd.wait()
        plsc.subcore_barrier()
        cg = [pltpu.async_copy(hpre.at[idx_v.at[NCHUNK + j]],
                               buf_v.at[pl.ds(j * ECH, ECH)], sem)
              for j in range(NCHUNK)]
        cw = []
        for j in range(NCHUNK):
            cg[j].wait()
            cw.append(pltpu.async_copy(
                buf_v.at[pl.ds(j * ECH, ECH)],
                xnext.at[pl.ds(eb + j * ECH, ECH)], sem))
        for c in cw:
            c.wait()

    return k


def _tc_layer(fin, fout, first):
    def body(x_ref, h_ref, ep_ref, w_ref, r_ref, b_ref, msgs_ref, root_ref):
        X = x_ref[...]
        H = h_ref[...]
        if not first:
            X = jnp.maximum(X, 0.0)
            H = jnp.maximum(H, 0.0)
        acc = ep_ref[:, 0:1] * jnp.dot(X, w_ref[0],
                                       preferred_element_type=jnp.float32)
        for c in range(1, 5):
            acc += ep_ref[:, c:c + 1] * jnp.dot(
                X, w_ref[c], preferred_element_type=jnp.float32)
        msgs_ref[...] = acc
        root_ref[...] = jnp.dot(H, r_ref[...],
                                preferred_element_type=jnp.float32) + b_ref[...]

    return pl.pallas_call(
        body,
        out_shape=(jax.ShapeDtypeStruct((N_EDGES, fout), jnp.float32),
                   jax.ShapeDtypeStruct((N_NODES, fout), jnp.float32)),
    )


def _tc_last():
    def body(x_ref, h_ref, ep_ref, w_ref, r_ref, b_ref, dst_ref,
             ia_ref, ib_ref, wd_ref, bd_ref, o_ref):
        X = jnp.maximum(x_ref[...], 0.0)
        H = jnp.maximum(h_ref[...], 0.0)
        acc = ep_ref[:, 0:1] * jnp.dot(X, w_ref[0],
                                       preferred_element_type=jnp.float32)
        for c in range(1, 5):
            acc += ep_ref[:, c:c + 1] * jnp.dot(
                X, w_ref[c], preferred_element_type=jnp.float32)
        root = jnp.dot(H, r_ref[...],
                       preferred_element_type=jnp.float32) + b_ref[...]
        ids = lax.broadcasted_iota(jnp.int32, (N_NODES, N_EDGES), 0)
        oh = jnp.where(dst_ref[...] == ids, 1.0, 0.0)
        hpre = jnp.dot(oh, acc, preferred_element_type=jnp.float32) + root
        z = jnp.dot(jnp.maximum(hpre, 0.0), wd_ref[...],
                    preferred_element_type=jnp.float32) + bd_ref[...]
        z = jnp.maximum(z, 0.0)
        idn = lax.broadcasted_iota(jnp.int32, (N_PREF, N_NODES), 1)
        pd = (jnp.where(ib_ref[...] == idn, 1.0, 0.0)
              - jnp.where(ia_ref[...] == idn, 1.0, 0.0))
        o_ref[...] = jnp.dot(pd, z, preferred_element_type=jnp.float32)

    return pl.pallas_call(
        body, out_shape=jax.ShapeDtypeStruct((N_PREF, 8), jnp.float32))


def kernel(x, a_indices, e, i, idx_a, idx_b,
           Wk1, bk1, R1, b1, Wk2, bk2, R2, b2, Wk3, bk3, R3, b3,
           Wk4, bk4, R4, b4, Wk5, bk5, R5, b5, Wd, bd):
    del i
    x32 = x.astype(jnp.float32)[:, :32]
    e = e.astype(jnp.float32)
    src3 = a_indices[:, 0].reshape(NSUB, NCHUNK, ECH)
    dst3 = a_indices[:, 1].reshape(NSUB, NCHUNK, ECH)
    ep = jnp.concatenate([e, jnp.ones((N_EDGES, 1), jnp.float32)], axis=1)

    Wks = [Wk1, Wk2, Wk3, Wk4, Wk5]
    bks = [bk1, bk2, bk3, bk4, bk5]
    Rs = [R1, R2, R3, R4, R5]
    bs = [b1, b2, b3, b4, b5]
    W5s, b2ds = [], []
    for l in range(5):
        fin, fout = FINS[l], FOUTS[l]
        W5s.append(jnp.concatenate(
            [Wks[l].reshape(4, fin, fout), bks[l].reshape(1, fin, fout)],
            axis=0))
        b2ds.append(bs[l].reshape(1, fout))

    idx3 = jnp.concatenate([dst3, src3], axis=1)

    X = _sc_gather(N_NODES, 32)(x32, src3)
    h = x32
    for l in range(4):
        msgs, root = _tc_layer(FINS[l], FOUTS[l], first=(l == 0))(
            X, h, ep, W5s[l], Rs[l], b2ds[l])
        h, X = _sc_layer(FOUTS[l])(msgs, idx3, root)

    return _tc_last()(
        X, h, ep, W5s[4], Rs[4], b2ds[4],
        a_indices[:, 1].reshape(1, N_EDGES),
        idx_a.reshape(N_PREF, 1), idx_b.reshape(N_PREF, 1),
        Wd, bd.reshape(1, 8))
